# Initial kernel scaffold; baseline (speedup 1.0000x reference)
import jax
import jax.numpy as jnp
from jax.experimental import pallas as pl


def kernel(nodes, edges, senders, receivers, W_enc1, b_enc1, W_enc2, b_enc2, W_dec1, b_dec1, W_dec2, b_dec2, W_mu, b_mu, W_lv, b_lv, W_p1, b_p1, W_p2, b_p2):
    raise NotImplementedError("write your pallas kernel here")


# speedup vs baseline: 63.0565x; 63.0565x over previous
import functools

import jax
import jax.numpy as jnp
from jax import lax
from jax.experimental import pallas as pl

N = 10000
D = 128
BLOCK = 2000
GRID = N // BLOCK


def _fused_kernel(nodes_ref, w1_ref, b1_ref, w2_ref, b2_ref,
                  wd1_ref, bd1_ref, wd2_ref, bd2_ref,
                  wmu_ref, bmu_ref, wlv_ref, blv_ref,
                  wp1_ref, bp1_ref, wp2_ref, bp2_ref, eps_ref,
                  assign_ref, coarse_ref, mu_ref, lv_ref, pred_ref):
    i = pl.program_id(0)

    x = nodes_ref[...]
    h1 = jnp.maximum(
        jnp.dot(x, w1_ref[...], preferred_element_type=jnp.float32)
        + b1_ref[...], 0.0)
    h = jnp.dot(h1, w2_ref[...], preferred_element_type=jnp.float32) \
        + b2_ref[...]
    a = jnp.maximum(
        jnp.dot(h, wd1_ref[...], preferred_element_type=jnp.float32)
        + bd1_ref[...], 0.0)
    logits = jnp.dot(a, wd2_ref[...], preferred_element_type=jnp.float32) \
        + bd2_ref[...]
    m = jnp.max(logits, axis=-1, keepdims=True)
    e = jnp.exp(logits - m)
    assign = e / jnp.sum(e, axis=-1, keepdims=True)
    assign_ref[...] = assign

    partial = lax.dot_general(assign, h, (((0,), (0,)), ((), ())),
                              preferred_element_type=jnp.float32)

    @pl.when(i == 0)
    def _():
        coarse_ref[...] = partial

    @pl.when(i > 0)
    def _():
        coarse_ref[...] += partial

    @pl.when(i == GRID - 1)
    def _():
        macro = jnp.mean(coarse_ref[...], axis=0, keepdims=True)
        mu = jnp.dot(macro, wmu_ref[...],
                     preferred_element_type=jnp.float32) + bmu_ref[...]
        lv = jnp.dot(macro, wlv_ref[...],
                     preferred_element_type=jnp.float32) + blv_ref[...]
        std = jnp.exp(0.5 * lv)
        z = mu + eps_ref[...] * std
        p = jnp.maximum(
            jnp.dot(z, wp1_ref[...], preferred_element_type=jnp.float32)
            + bp1_ref[...], 0.0)
        pred = jnp.dot(p, wp2_ref[...],
                       preferred_element_type=jnp.float32) + bp2_ref[...]
        mu_ref[...] = mu
        lv_ref[...] = lv
        pred_ref[...] = pred


@functools.partial(jax.jit, static_argnames=("interpret",))
def _run(nodes, W_enc1, b_enc1, W_enc2, b_enc2,
         W_dec1, b_dec1, W_dec2, b_dec2,
         W_mu, b_mu, W_lv, b_lv,
         W_p1, b_p1, W_p2, b_p2, eps, interpret=False):
    full = lambda *shape: pl.BlockSpec(shape, lambda i: (0,) * len(shape))
    out = pl.pallas_call(
        _fused_kernel,
        grid=(GRID,),
        in_specs=[
            pl.BlockSpec((BLOCK, D), lambda i: (i, 0)),
            full(128, 128), full(1, 128),
            full(128, 128), full(1, 128),
            full(128, 32), full(1, 32),
            full(32, 64), full(1, 64),
            full(128, 64), full(1, 64),
            full(128, 64), full(1, 64),
            full(64, 32), full(1, 32),
            full(32, 1), full(1, 1),
            full(1, 64),
        ],
        out_specs=[
            pl.BlockSpec((BLOCK, 64), lambda i: (i, 0)),
            full(64, 128),
            full(1, 64), full(1, 64), full(1, 1),
        ],
        out_shape=[
            jax.ShapeDtypeStruct((N, 64), jnp.float32),
            jax.ShapeDtypeStruct((64, 128), jnp.float32),
            jax.ShapeDtypeStruct((1, 64), jnp.float32),
            jax.ShapeDtypeStruct((1, 64), jnp.float32),
            jax.ShapeDtypeStruct((1, 1), jnp.float32),
        ],
        interpret=interpret,
    )(nodes, W_enc1, b_enc1.reshape(1, -1), W_enc2, b_enc2.reshape(1, -1),
      W_dec1, b_dec1.reshape(1, -1), W_dec2, b_dec2.reshape(1, -1),
      W_mu, b_mu.reshape(1, -1), W_lv, b_lv.reshape(1, -1),
      W_p1, b_p1.reshape(1, -1), W_p2, b_p2.reshape(1, -1), eps)
    assignments, coarse_nodes, mu, lv, pred = out
    return (mu.reshape(-1), lv.reshape(-1), pred.reshape(-1),
            assignments, coarse_nodes)


def kernel(nodes, edges, senders, receivers,
           W_enc1, b_enc1, W_enc2, b_enc2,
           W_dec1, b_dec1, W_dec2, b_dec2,
           W_mu, b_mu, W_lv, b_lv,
           W_p1, b_p1, W_p2, b_p2):
    eps = jax.random.normal(jax.random.PRNGKey(0), (1, 64), jnp.float32)
    return _run(nodes, W_enc1, b_enc1, W_enc2, b_enc2,
                W_dec1, b_dec1, W_dec2, b_dec2,
                W_mu, b_mu, W_lv, b_lv,
                W_p1, b_p1, W_p2, b_p2, eps)


# speedup vs baseline: 64.1128x; 1.0168x over previous
import functools

import jax
import jax.numpy as jnp
from jax import lax
from jax.experimental import pallas as pl

N = 10000
D = 128
BLOCK = 2000
GRID = N // BLOCK

_EPS = jax.random.normal(jax.random.PRNGKey(0), (64,), jnp.float32).reshape(1, 64)


def _fused_kernel(nodes_ref, w1_ref, b1_ref, w2_ref, b2_ref,
                  wd1_ref, bd1_ref, wd2_ref, bd2_ref,
                  wmu_ref, bmu_ref, wlv_ref, blv_ref,
                  wp1_ref, bp1_ref, wp2_ref, bp2_ref, eps_ref,
                  assign_ref, coarse_ref, mu_ref, lv_ref, pred_ref):
    i = pl.program_id(0)

    x = nodes_ref[...]
    h1 = jnp.maximum(
        jnp.dot(x, w1_ref[...], preferred_element_type=jnp.float32)
        + b1_ref[...], 0.0)
    h = jnp.dot(h1, w2_ref[...], preferred_element_type=jnp.float32) \
        + b2_ref[...]
    a = jnp.maximum(
        jnp.dot(h, wd1_ref[...], preferred_element_type=jnp.float32)
        + bd1_ref[...], 0.0)
    logits = jnp.dot(a, wd2_ref[...], preferred_element_type=jnp.float32) \
        + bd2_ref[...]
    m = jnp.max(logits, axis=-1, keepdims=True)
    e = jnp.exp(logits - m)
    assign = e / jnp.sum(e, axis=-1, keepdims=True)
    assign_ref[...] = assign

    partial = lax.dot_general(assign, h, (((0,), (0,)), ((), ())),
                              preferred_element_type=jnp.float32)

    @pl.when(i == 0)
    def _():
        coarse_ref[...] = partial

    @pl.when(i > 0)
    def _():
        coarse_ref[...] += partial

    @pl.when(i == GRID - 1)
    def _():
        macro = jnp.mean(coarse_ref[...], axis=0, keepdims=True)
        mu = jnp.dot(macro, wmu_ref[...],
                     preferred_element_type=jnp.float32) + bmu_ref[...]
        lv = jnp.dot(macro, wlv_ref[...],
                     preferred_element_type=jnp.float32) + blv_ref[...]
        std = jnp.exp(0.5 * lv)
        z = mu + eps_ref[...] * std
        p = jnp.maximum(
            jnp.dot(z, wp1_ref[...], preferred_element_type=jnp.float32)
            + bp1_ref[...], 0.0)
        pred = jnp.dot(p, wp2_ref[...],
                       preferred_element_type=jnp.float32) + bp2_ref[...]
        mu_ref[...] = mu
        lv_ref[...] = lv
        pred_ref[...] = pred


@functools.partial(jax.jit, static_argnames=("interpret",))
def _run(nodes, W_enc1, b_enc1, W_enc2, b_enc2,
         W_dec1, b_dec1, W_dec2, b_dec2,
         W_mu, b_mu, W_lv, b_lv,
         W_p1, b_p1, W_p2, b_p2, eps, interpret=False):
    full = lambda *shape: pl.BlockSpec(shape, lambda i: (0,) * len(shape))
    out = pl.pallas_call(
        _fused_kernel,
        grid=(GRID,),
        in_specs=[
            pl.BlockSpec((BLOCK, D), lambda i: (i, 0)),
            full(128, 128), full(1, 128),
            full(128, 128), full(1, 128),
            full(128, 32), full(1, 32),
            full(32, 64), full(1, 64),
            full(128, 64), full(1, 64),
            full(128, 64), full(1, 64),
            full(64, 32), full(1, 32),
            full(32, 1), full(1, 1),
            full(1, 64),
        ],
        out_specs=[
            pl.BlockSpec((BLOCK, 64), lambda i: (i, 0)),
            full(64, 128),
            full(1, 64), full(1, 64), full(1, 1),
        ],
        out_shape=[
            jax.ShapeDtypeStruct((N, 64), jnp.float32),
            jax.ShapeDtypeStruct((64, 128), jnp.float32),
            jax.ShapeDtypeStruct((1, 64), jnp.float32),
            jax.ShapeDtypeStruct((1, 64), jnp.float32),
            jax.ShapeDtypeStruct((1, 1), jnp.float32),
        ],
        interpret=interpret,
    )(nodes, W_enc1, b_enc1.reshape(1, -1), W_enc2, b_enc2.reshape(1, -1),
      W_dec1, b_dec1.reshape(1, -1), W_dec2, b_dec2.reshape(1, -1),
      W_mu, b_mu.reshape(1, -1), W_lv, b_lv.reshape(1, -1),
      W_p1, b_p1.reshape(1, -1), W_p2, b_p2.reshape(1, -1), eps)
    assignments, coarse_nodes, mu, lv, pred = out
    return (mu.reshape(-1), lv.reshape(-1), pred.reshape(-1),
            assignments, coarse_nodes)


def kernel(nodes, edges, senders, receivers,
           W_enc1, b_enc1, W_enc2, b_enc2,
           W_dec1, b_dec1, W_dec2, b_dec2,
           W_mu, b_mu, W_lv, b_lv,
           W_p1, b_p1, W_p2, b_p2):
    eps = _EPS
    return _run(nodes, W_enc1, b_enc1, W_enc2, b_enc2,
                W_dec1, b_dec1, W_dec2, b_dec2,
                W_mu, b_mu, W_lv, b_lv,
                W_p1, b_p1, W_p2, b_p2, eps)
